# baseline (device time: 159387 ns/iter reference)
import functools

import jax
import jax.numpy as jnp
from jax import lax
from jax.experimental import pallas as pl
from jax.experimental.pallas import tpu as pltpu

N_DEV = 4
M = 4096
K_SHARD = 1024
N_OUT = 2048
HALF = N_OUT // 2
CHUNK = M // N_DEV


def kernel(x, w_mat):
    def body(
        x_hbm,
        w_ref,
        out_hbm,
        xs_ref,
        xsb_ref,
        wb_ref,
        acc_r_ref,
        acc_l_ref,
        rsr_ref,
        rsl_ref,
        qr_ref,
        ql_ref,
        stage_r_ref,
        stage_l_ref,
        amax_ref,
        copy_sem,
        out_sem_r,
        out_sem_l,
        rs_send_r,
        rs_recv_r,
        rs_send_l,
        rs_recv_l,
        ax_send,
        ax_recv,
        ag_send_r,
        ag_recv_r,
        ag_send_l,
        ag_recv_l,
    ):
        my = lax.axis_index("i")
        right = (my + 1) % N_DEV
        left = (my + N_DEV - 1) % N_DEV

        barrier_sem = pltpu.get_barrier_semaphore()
        for nbr in (left, right):
            pl.semaphore_signal(
                barrier_sem, inc=1,
                device_id=(nbr,), device_id_type=pl.DeviceIdType.MESH,
            )
        pl.semaphore_wait(barrier_sem, 2)

        wb_ref[...] = w_ref[...].astype(jnp.bfloat16)

        def load_x(c):
            cp = pltpu.make_async_copy(
                x_hbm.at[pl.ds(c * CHUNK, CHUNK), :], xs_ref, copy_sem
            )
            cp.start()
            cp.wait()
            xsb_ref[...] = xs_ref[...].astype(jnp.bfloat16)

        def dot_half(lo):
            return jnp.dot(
                xsb_ref[...],
                wb_ref[:, lo : lo + HALF],
                preferred_element_type=jnp.float32,
            )

        def rs_rdma(h):
            rdma_r = pltpu.make_async_remote_copy(
                src_ref=acc_r_ref,
                dst_ref=rsr_ref.at[h],
                send_sem=rs_send_r.at[h],
                recv_sem=rs_recv_r.at[h],
                device_id=(right,),
                device_id_type=pl.DeviceIdType.MESH,
            )
            rdma_l = pltpu.make_async_remote_copy(
                src_ref=acc_l_ref,
                dst_ref=rsl_ref.at[h],
                send_sem=rs_send_l.at[h],
                recv_sem=rs_recv_l.at[h],
                device_id=(left,),
                device_id_type=pl.DeviceIdType.MESH,
            )
            return rdma_r, rdma_l

        load_x(my)
        acc_r_ref[...] = dot_half(0).astype(jnp.bfloat16)
        rdma_r, rdma_l = rs_rdma(0)
        rdma_r.start()
        acc_l_ref[...] = dot_half(HALF).astype(jnp.bfloat16)
        rdma_l.start()
        for h in range(N_DEV - 1):
            load_x((my - h - 1) % N_DEV)
            p_r = dot_half(0)
            if h != 1:
                load_x((my + h + 1) % N_DEV)
            p_l = dot_half(HALF)
            rdma_r.wait()
            acc_r_ref[...] = (rsr_ref[h].astype(jnp.float32) + p_r).astype(
                jnp.bfloat16
            )
            if h + 1 < N_DEV - 1:
                next_r, next_l = rs_rdma(h + 1)
                next_r.start()
            rdma_l.wait()
            acc_l_ref[...] = (rsl_ref[h].astype(jnp.float32) + p_l).astype(
                jnp.bfloat16
            )
            if h + 1 < N_DEV - 1:
                next_l.start()
                rdma_r, rdma_l = next_r, next_l

        own_r = (my + 1) % N_DEV
        own_l = (my + N_DEV - 1) % N_DEV

        my_max = jnp.maximum(
            jnp.max(jnp.abs(acc_r_ref[...].astype(jnp.float32))),
            jnp.max(jnp.abs(acc_l_ref[...].astype(jnp.float32))),
        )
        amax_ref[pl.ds(my, 1)] = jnp.full((1, 8, 128), my_max, jnp.float32)
        ax_rdmas = []
        for k in range(1, N_DEV):
            rdma = pltpu.make_async_remote_copy(
                src_ref=amax_ref.at[pl.ds(my, 1)],
                dst_ref=amax_ref.at[pl.ds(my, 1)],
                send_sem=ax_send.at[k - 1],
                recv_sem=ax_recv.at[k - 1],
                device_id=((my + k) % N_DEV,),
                device_id_type=pl.DeviceIdType.MESH,
            )
            rdma.start()
            ax_rdmas.append(rdma)
        for rdma in ax_rdmas:
            rdma.wait()
        amax = jnp.max(amax_ref[...])
        scale = amax / 127.0

        def quantize(v):
            return jnp.clip(jnp.round(v / scale), -127.0, 127.0).astype(
                jnp.int8
            )

        qr_ref[pl.ds(own_r, 1)] = quantize(acc_r_ref[...].astype(jnp.float32))[
            None
        ]
        ql_ref[pl.ds(own_l, 1)] = quantize(acc_l_ref[...].astype(jnp.float32))[
            None
        ]

        def store_half(q_ref_, stage_ref, sem, c, col_lo):
            stage_ref[...] = q_ref_[pl.ds(c, 1)][0].astype(jnp.float32) * scale
            cp = pltpu.make_async_copy(
                stage_ref,
                out_hbm.at[pl.ds(c * CHUNK, CHUNK), pl.ds(col_lo, HALF)],
                sem,
            )
            cp.start()
            return cp

        for g in range(N_DEV - 1):
            rdma_r = pltpu.make_async_remote_copy(
                src_ref=qr_ref.at[pl.ds((my + 1 - g) % N_DEV, 1)],
                dst_ref=qr_ref.at[pl.ds((my + 1 - g) % N_DEV, 1)],
                send_sem=ag_send_r.at[g],
                recv_sem=ag_recv_r.at[g],
                device_id=(right,),
                device_id_type=pl.DeviceIdType.MESH,
            )
            rdma_l = pltpu.make_async_remote_copy(
                src_ref=ql_ref.at[pl.ds((my - 1 + g) % N_DEV, 1)],
                dst_ref=ql_ref.at[pl.ds((my - 1 + g) % N_DEV, 1)],
                send_sem=ag_send_l.at[g],
                recv_sem=ag_recv_l.at[g],
                device_id=(left,),
                device_id_type=pl.DeviceIdType.MESH,
            )
            rdma_r.start()
            rdma_l.start()
            cp_r = store_half(
                qr_ref, stage_r_ref, out_sem_r, (my + 1 - g) % N_DEV, 0
            )
            cp_l = store_half(
                ql_ref, stage_l_ref, out_sem_l, (my - 1 + g) % N_DEV, HALF
            )
            cp_r.wait()
            cp_l.wait()
            rdma_r.wait()
            rdma_l.wait()
        cp_r = store_half(qr_ref, stage_r_ref, out_sem_r, (my - 2) % N_DEV, 0)
        cp_l = store_half(
            ql_ref, stage_l_ref, out_sem_l, (my + 2) % N_DEV, HALF
        )
        cp_r.wait()
        cp_l.wait()

        @functools.partial(
            pl.run_scoped, second_barrier=pltpu.SemaphoreType.REGULAR
        )
        def _(second_barrier):
            for nbr in (left, right):
                pl.semaphore_signal(
                    second_barrier, inc=1,
                    device_id=(nbr,), device_id_type=pl.DeviceIdType.MESH,
                )
            pl.semaphore_wait(second_barrier, 2)

    return pl.pallas_call(
        body,
        out_shape=jax.ShapeDtypeStruct((M, N_OUT), jnp.float32),
        in_specs=[
            pl.BlockSpec(memory_space=pl.ANY),
            pl.BlockSpec(memory_space=pltpu.VMEM),
        ],
        out_specs=pl.BlockSpec(memory_space=pl.ANY),
        scratch_shapes=[
            pltpu.VMEM((CHUNK, K_SHARD), jnp.float32),
            pltpu.VMEM((CHUNK, K_SHARD), jnp.bfloat16),
            pltpu.VMEM((K_SHARD, N_OUT), jnp.bfloat16),
            pltpu.VMEM((CHUNK, HALF), jnp.bfloat16),
            pltpu.VMEM((CHUNK, HALF), jnp.bfloat16),
            pltpu.VMEM((N_DEV - 1, CHUNK, HALF), jnp.bfloat16),
            pltpu.VMEM((N_DEV - 1, CHUNK, HALF), jnp.bfloat16),
            pltpu.VMEM((N_DEV, CHUNK, HALF), jnp.int8),
            pltpu.VMEM((N_DEV, CHUNK, HALF), jnp.int8),
            pltpu.VMEM((CHUNK, HALF), jnp.float32),
            pltpu.VMEM((CHUNK, HALF), jnp.float32),
            pltpu.VMEM((N_DEV, 8, 128), jnp.float32),
            pltpu.SemaphoreType.DMA,
            pltpu.SemaphoreType.DMA,
            pltpu.SemaphoreType.DMA,
            pltpu.SemaphoreType.DMA((N_DEV - 1,)),
            pltpu.SemaphoreType.DMA((N_DEV - 1,)),
            pltpu.SemaphoreType.DMA((N_DEV - 1,)),
            pltpu.SemaphoreType.DMA((N_DEV - 1,)),
            pltpu.SemaphoreType.DMA((N_DEV - 1,)),
            pltpu.SemaphoreType.DMA((N_DEV - 1,)),
            pltpu.SemaphoreType.DMA((N_DEV - 1,)),
            pltpu.SemaphoreType.DMA((N_DEV - 1,)),
            pltpu.SemaphoreType.DMA((N_DEV - 1,)),
            pltpu.SemaphoreType.DMA((N_DEV - 1,)),
        ],
        compiler_params=pltpu.CompilerParams(
            collective_id=0,
            vmem_limit_bytes=128 * 1024 * 1024,
        ),
    )(x, w_mat)


# device time: 153408 ns/iter; 1.0390x vs baseline; 1.0390x over previous
import functools

import jax
import jax.numpy as jnp
from jax import lax
from jax.experimental import pallas as pl
from jax.experimental.pallas import tpu as pltpu

N_DEV = 4
M = 4096
K_SHARD = 1024
N_OUT = 2048
HALF = N_OUT // 2
CHUNK = M // N_DEV
SUB = CHUNK // 2


def kernel(x, w_mat):
    def body(
        x_hbm,
        w_ref,
        out_hbm,
        xs_ref,
        xsb_ref,
        wb_ref,
        acc_r_ref,
        acc_l_ref,
        rsr_ref,
        rsl_ref,
        qr_ref,
        ql_ref,
        stage_r_ref,
        stage_l_ref,
        amax_ref,
        copy_sem,
        out_sem_r,
        out_sem_l,
        rs_send_r,
        rs_recv_r,
        rs_send_l,
        rs_recv_l,
        ax_send,
        ax_recv,
        ag_send_r,
        ag_recv_r,
        ag_send_l,
        ag_recv_l,
    ):
        my = lax.axis_index("i")
        right = (my + 1) % N_DEV
        left = (my + N_DEV - 1) % N_DEV

        barrier_sem = pltpu.get_barrier_semaphore()
        for nbr in (left, right):
            pl.semaphore_signal(
                barrier_sem, inc=1,
                device_id=(nbr,), device_id_type=pl.DeviceIdType.MESH,
            )
        pl.semaphore_wait(barrier_sem, 2)

        wb_ref[...] = w_ref[...].astype(jnp.bfloat16)

        def load_x(c):
            cp = pltpu.make_async_copy(
                x_hbm.at[pl.ds(c * CHUNK, CHUNK), :], xs_ref, copy_sem
            )
            cp.start()
            cp.wait()
            xsb_ref[...] = xs_ref[...].astype(jnp.bfloat16)

        def dot_half(lo):
            return jnp.dot(
                xsb_ref[...],
                wb_ref[:, lo : lo + HALF],
                preferred_element_type=jnp.float32,
            )

        def rs_rdma(h, s, acc, rs_ref_, send_sems, recv_sems, dev):
            rows = pl.ds(s * SUB, SUB)
            return pltpu.make_async_remote_copy(
                src_ref=acc.at[rows],
                dst_ref=rs_ref_.at[h, rows],
                send_sem=send_sems.at[h, s],
                recv_sem=recv_sems.at[h, s],
                device_id=(dev,),
                device_id_type=pl.DeviceIdType.MESH,
            )

        def rs_rdma_r(h, s):
            return rs_rdma(h, s, acc_r_ref, rsr_ref, rs_send_r, rs_recv_r, right)

        def rs_rdma_l(h, s):
            return rs_rdma(h, s, acc_l_ref, rsl_ref, rs_send_l, rs_recv_l, left)

        def dot_block(row_lo, col_lo):
            return jnp.dot(
                xsb_ref[row_lo : row_lo + SUB, :],
                wb_ref[:, col_lo : col_lo + HALF],
                preferred_element_type=jnp.float32,
            )

        load_x(my)
        acc_r_ref[0:SUB] = dot_block(0, 0).astype(jnp.bfloat16)
        cur = [[rs_rdma_r(0, 0), None], [None, None]]
        cur[0][0].start()
        acc_l_ref[0:SUB] = dot_block(0, HALF).astype(jnp.bfloat16)
        cur[1][0] = rs_rdma_l(0, 0)
        cur[1][0].start()
        acc_r_ref[SUB:CHUNK] = dot_block(SUB, 0).astype(jnp.bfloat16)
        cur[0][1] = rs_rdma_r(0, 1)
        cur[0][1].start()
        acc_l_ref[SUB:CHUNK] = dot_block(SUB, HALF).astype(jnp.bfloat16)
        cur[1][1] = rs_rdma_l(0, 1)
        cur[1][1].start()

        for h in range(N_DEV - 1):
            load_x((my - h - 1) % N_DEV)
            p_r = dot_half(0)
            if h != 1:
                load_x((my + h + 1) % N_DEV)
            p_l = dot_half(HALF)
            ps = [p_r, p_l]
            accs = [acc_r_ref, acc_l_ref]
            rs_refs = [rsr_ref, rsl_ref]
            mk = [rs_rdma_r, rs_rdma_l]
            nxt = [[None, None], [None, None]]
            last = h + 1 == N_DEV - 1
            for s in range(2):
                rows = slice(s * SUB, (s + 1) * SUB)
                for d in range(2):
                    cur[d][s].wait()
                    accs[d][rows] = (
                        rs_refs[d][h, rows].astype(jnp.float32) + ps[d][rows]
                    ).astype(jnp.bfloat16)
                    if not last:
                        nxt[d][s] = mk[d](h + 1, s)
                        nxt[d][s].start()
            cur = nxt

        own_r = (my + 1) % N_DEV
        own_l = (my + N_DEV - 1) % N_DEV

        my_max = jnp.maximum(
            jnp.max(jnp.abs(acc_r_ref[...].astype(jnp.float32))),
            jnp.max(jnp.abs(acc_l_ref[...].astype(jnp.float32))),
        )
        amax_ref[pl.ds(my, 1)] = jnp.full((1, 8, 128), my_max, jnp.float32)
        ax_rdmas = []
        for k in range(1, N_DEV):
            rdma = pltpu.make_async_remote_copy(
                src_ref=amax_ref.at[pl.ds(my, 1)],
                dst_ref=amax_ref.at[pl.ds(my, 1)],
                send_sem=ax_send.at[k - 1],
                recv_sem=ax_recv.at[k - 1],
                device_id=((my + k) % N_DEV,),
                device_id_type=pl.DeviceIdType.MESH,
            )
            rdma.start()
            ax_rdmas.append(rdma)
        for rdma in ax_rdmas:
            rdma.wait()
        amax = jnp.max(amax_ref[...])
        scale = amax / 127.0

        def quantize(v):
            return jnp.clip(jnp.round(v / scale), -127.0, 127.0).astype(
                jnp.int8
            )

        qr_ref[pl.ds(own_r, 1)] = quantize(acc_r_ref[...].astype(jnp.float32))[
            None
        ]
        ql_ref[pl.ds(own_l, 1)] = quantize(acc_l_ref[...].astype(jnp.float32))[
            None
        ]

        def store_half(q_ref_, stage_ref, sem, c, col_lo):
            stage_ref[...] = q_ref_[pl.ds(c, 1)][0].astype(jnp.float32) * scale
            cp = pltpu.make_async_copy(
                stage_ref,
                out_hbm.at[pl.ds(c * CHUNK, CHUNK), pl.ds(col_lo, HALF)],
                sem,
            )
            cp.start()
            return cp

        for g in range(N_DEV - 1):
            rdma_r = pltpu.make_async_remote_copy(
                src_ref=qr_ref.at[pl.ds((my + 1 - g) % N_DEV, 1)],
                dst_ref=qr_ref.at[pl.ds((my + 1 - g) % N_DEV, 1)],
                send_sem=ag_send_r.at[g],
                recv_sem=ag_recv_r.at[g],
                device_id=(right,),
                device_id_type=pl.DeviceIdType.MESH,
            )
            rdma_l = pltpu.make_async_remote_copy(
                src_ref=ql_ref.at[pl.ds((my - 1 + g) % N_DEV, 1)],
                dst_ref=ql_ref.at[pl.ds((my - 1 + g) % N_DEV, 1)],
                send_sem=ag_send_l.at[g],
                recv_sem=ag_recv_l.at[g],
                device_id=(left,),
                device_id_type=pl.DeviceIdType.MESH,
            )
            rdma_r.start()
            rdma_l.start()
            cp_r = store_half(
                qr_ref, stage_r_ref, out_sem_r, (my + 1 - g) % N_DEV, 0
            )
            cp_l = store_half(
                ql_ref, stage_l_ref, out_sem_l, (my - 1 + g) % N_DEV, HALF
            )
            cp_r.wait()
            cp_l.wait()
            rdma_r.wait()
            rdma_l.wait()
        cp_r = store_half(qr_ref, stage_r_ref, out_sem_r, (my - 2) % N_DEV, 0)
        cp_l = store_half(
            ql_ref, stage_l_ref, out_sem_l, (my + 2) % N_DEV, HALF
        )
        cp_r.wait()
        cp_l.wait()

        @functools.partial(
            pl.run_scoped, second_barrier=pltpu.SemaphoreType.REGULAR
        )
        def _(second_barrier):
            for nbr in (left, right):
                pl.semaphore_signal(
                    second_barrier, inc=1,
                    device_id=(nbr,), device_id_type=pl.DeviceIdType.MESH,
                )
            pl.semaphore_wait(second_barrier, 2)

    return pl.pallas_call(
        body,
        out_shape=jax.ShapeDtypeStruct((M, N_OUT), jnp.float32),
        in_specs=[
            pl.BlockSpec(memory_space=pl.ANY),
            pl.BlockSpec(memory_space=pltpu.VMEM),
        ],
        out_specs=pl.BlockSpec(memory_space=pl.ANY),
        scratch_shapes=[
            pltpu.VMEM((CHUNK, K_SHARD), jnp.float32),
            pltpu.VMEM((CHUNK, K_SHARD), jnp.bfloat16),
            pltpu.VMEM((K_SHARD, N_OUT), jnp.bfloat16),
            pltpu.VMEM((CHUNK, HALF), jnp.bfloat16),
            pltpu.VMEM((CHUNK, HALF), jnp.bfloat16),
            pltpu.VMEM((N_DEV - 1, CHUNK, HALF), jnp.bfloat16),
            pltpu.VMEM((N_DEV - 1, CHUNK, HALF), jnp.bfloat16),
            pltpu.VMEM((N_DEV, CHUNK, HALF), jnp.int8),
            pltpu.VMEM((N_DEV, CHUNK, HALF), jnp.int8),
            pltpu.VMEM((CHUNK, HALF), jnp.float32),
            pltpu.VMEM((CHUNK, HALF), jnp.float32),
            pltpu.VMEM((N_DEV, 8, 128), jnp.float32),
            pltpu.SemaphoreType.DMA,
            pltpu.SemaphoreType.DMA,
            pltpu.SemaphoreType.DMA,
            pltpu.SemaphoreType.DMA((N_DEV - 1, 2)),
            pltpu.SemaphoreType.DMA((N_DEV - 1, 2)),
            pltpu.SemaphoreType.DMA((N_DEV - 1, 2)),
            pltpu.SemaphoreType.DMA((N_DEV - 1, 2)),
            pltpu.SemaphoreType.DMA((N_DEV - 1,)),
            pltpu.SemaphoreType.DMA((N_DEV - 1,)),
            pltpu.SemaphoreType.DMA((N_DEV - 1,)),
            pltpu.SemaphoreType.DMA((N_DEV - 1,)),
            pltpu.SemaphoreType.DMA((N_DEV - 1,)),
            pltpu.SemaphoreType.DMA((N_DEV - 1,)),
        ],
        compiler_params=pltpu.CompilerParams(
            collective_id=0,
            vmem_limit_bytes=128 * 1024 * 1024,
        ),
    )(x, w_mat)


# device time: 149868 ns/iter; 1.0635x vs baseline; 1.0236x over previous
import functools

import jax
import jax.numpy as jnp
from jax import lax
from jax.experimental import pallas as pl
from jax.experimental.pallas import tpu as pltpu

N_DEV = 4
M = 4096
K_SHARD = 1024
N_OUT = 2048
HALF = N_OUT // 2
CHUNK = M // N_DEV
SUB = CHUNK // 2


def kernel(x, w_mat):
    def body(
        x_hbm,
        w_ref,
        out_hbm,
        xs_ref,
        xsb_ref,
        wb_ref,
        acc_r_ref,
        acc_l_ref,
        rsr_ref,
        rsl_ref,
        qr_ref,
        ql_ref,
        stage_r_ref,
        stage_l_ref,
        amax_ref,
        copy_sem,
        out_sem_r,
        out_sem_l,
        rs_send_r,
        rs_recv_r,
        rs_send_l,
        rs_recv_l,
        ax_send,
        ax_recv,
        ag_send_r,
        ag_recv_r,
        ag_send_l,
        ag_recv_l,
    ):
        my = lax.axis_index("i")
        right = (my + 1) % N_DEV
        left = (my + N_DEV - 1) % N_DEV

        barrier_sem = pltpu.get_barrier_semaphore()
        for nbr in (left, right):
            pl.semaphore_signal(
                barrier_sem, inc=1,
                device_id=(nbr,), device_id_type=pl.DeviceIdType.MESH,
            )
        pl.semaphore_wait(barrier_sem, 2)

        wb_ref[...] = w_ref[...].astype(jnp.bfloat16)

        def load_x(c):
            cp = pltpu.make_async_copy(
                x_hbm.at[pl.ds(c * CHUNK, CHUNK), :], xs_ref, copy_sem
            )
            cp.start()
            cp.wait()
            xsb_ref[...] = xs_ref[...].astype(jnp.bfloat16)

        def dot_half(lo):
            return jnp.dot(
                xsb_ref[...],
                wb_ref[:, lo : lo + HALF],
                preferred_element_type=jnp.float32,
            )

        def rs_rdma(h, s, acc, rs_ref_, send_sems, recv_sems, dev):
            rows = pl.ds(s * SUB, SUB)
            return pltpu.make_async_remote_copy(
                src_ref=acc.at[rows],
                dst_ref=rs_ref_.at[h, rows],
                send_sem=send_sems.at[h, s],
                recv_sem=recv_sems.at[h, s],
                device_id=(dev,),
                device_id_type=pl.DeviceIdType.MESH,
            )

        def rs_rdma_r(h, s):
            return rs_rdma(h, s, acc_r_ref, rsr_ref, rs_send_r, rs_recv_r, right)

        def rs_rdma_l(h, s):
            return rs_rdma(h, s, acc_l_ref, rsl_ref, rs_send_l, rs_recv_l, left)

        def dot_block(row_lo, col_lo):
            return jnp.dot(
                xsb_ref[row_lo : row_lo + SUB, :],
                wb_ref[:, col_lo : col_lo + HALF],
                preferred_element_type=jnp.float32,
            )

        load_x(my)
        acc_r_ref[0:SUB] = dot_block(0, 0).astype(jnp.bfloat16)
        cur = [[rs_rdma_r(0, 0), None], [None, None]]
        cur[0][0].start()
        acc_l_ref[0:SUB] = dot_block(0, HALF).astype(jnp.bfloat16)
        cur[1][0] = rs_rdma_l(0, 0)
        cur[1][0].start()
        acc_r_ref[SUB:CHUNK] = dot_block(SUB, 0).astype(jnp.bfloat16)
        cur[0][1] = rs_rdma_r(0, 1)
        cur[0][1].start()
        acc_l_ref[SUB:CHUNK] = dot_block(SUB, HALF).astype(jnp.bfloat16)
        cur[1][1] = rs_rdma_l(0, 1)
        cur[1][1].start()

        for h in range(N_DEV - 1):
            load_x((my - h - 1) % N_DEV)
            p_r = dot_half(0)
            if h != 1:
                load_x((my + h + 1) % N_DEV)
            p_l = dot_half(HALF)
            ps = [p_r, p_l]
            accs = [acc_r_ref, acc_l_ref]
            rs_refs = [rsr_ref, rsl_ref]
            mk = [rs_rdma_r, rs_rdma_l]
            nxt = [[None, None], [None, None]]
            last = h + 1 == N_DEV - 1
            maxes = []
            for s in range(2):
                rows = slice(s * SUB, (s + 1) * SUB)
                for d in range(2):
                    cur[d][s].wait()
                    v = rs_refs[d][h, rows].astype(jnp.float32) + ps[d][rows]
                    accs[d][rows] = v.astype(jnp.bfloat16)
                    if last:
                        maxes.append(jnp.max(jnp.abs(v)))
                    else:
                        nxt[d][s] = mk[d](h + 1, s)
                        nxt[d][s].start()
            cur = nxt

        own_r = (my + 1) % N_DEV
        own_l = (my + N_DEV - 1) % N_DEV

        my_max = functools.reduce(jnp.maximum, maxes)
        amax_ref[pl.ds(my, 1)] = jnp.full((1, 8, 128), my_max, jnp.float32)
        ax_rdmas = []
        for k in range(1, N_DEV):
            rdma = pltpu.make_async_remote_copy(
                src_ref=amax_ref.at[pl.ds(my, 1)],
                dst_ref=amax_ref.at[pl.ds(my, 1)],
                send_sem=ax_send.at[k - 1],
                recv_sem=ax_recv.at[k - 1],
                device_id=((my + k) % N_DEV,),
                device_id_type=pl.DeviceIdType.MESH,
            )
            rdma.start()
            ax_rdmas.append(rdma)
        for rdma in ax_rdmas:
            rdma.wait()
        amax = jnp.max(amax_ref[...])
        scale = amax / 127.0

        def quantize(v):
            return jnp.clip(jnp.round(v / scale), -127.0, 127.0).astype(
                jnp.int8
            )

        qr_ref[pl.ds(own_r, 1)] = quantize(acc_r_ref[...].astype(jnp.float32))[
            None
        ]
        ql_ref[pl.ds(own_l, 1)] = quantize(acc_l_ref[...].astype(jnp.float32))[
            None
        ]

        def store_half(q_ref_, stage_ref, sem, c, col_lo):
            stage_ref[...] = q_ref_[pl.ds(c, 1)][0].astype(jnp.float32) * scale
            cp = pltpu.make_async_copy(
                stage_ref,
                out_hbm.at[pl.ds(c * CHUNK, CHUNK), pl.ds(col_lo, HALF)],
                sem,
            )
            cp.start()
            return cp

        def ag_rdma(d, g, s):
            c = (my + 1 - g) % N_DEV if d == 0 else (my - 1 + g) % N_DEV
            q = qr_ref if d == 0 else ql_ref
            ss = ag_send_r if d == 0 else ag_send_l
            rs = ag_recv_r if d == 0 else ag_recv_l
            return pltpu.make_async_remote_copy(
                src_ref=q.at[pl.ds(c, 1), pl.ds(s * SUB, SUB)],
                dst_ref=q.at[pl.ds(c, 1), pl.ds(s * SUB, SUB)],
                send_sem=ss.at[g, s],
                recv_sem=rs.at[g, s],
                device_id=(right if d == 0 else left,),
                device_id_type=pl.DeviceIdType.MESH,
            )

        prev = [[ag_rdma(d, 0, s) for s in range(2)] for d in range(2)]
        for s in range(2):
            for d in range(2):
                prev[d][s].start()
        cp_r = store_half(qr_ref, stage_r_ref, out_sem_r, own_r, 0)
        cp_l = store_half(ql_ref, stage_l_ref, out_sem_l, own_l, HALF)
        cp_r.wait()
        cp_l.wait()
        for g in range(1, N_DEV - 1):
            nxt = [[None, None], [None, None]]
            for s in range(2):
                for d in range(2):
                    prev[d][s].wait()
                    nxt[d][s] = ag_rdma(d, g, s)
                    nxt[d][s].start()
            cp_r = store_half(
                qr_ref, stage_r_ref, out_sem_r, (my - g + 1) % N_DEV, 0
            )
            cp_l = store_half(
                ql_ref, stage_l_ref, out_sem_l, (my + g - 1) % N_DEV, HALF
            )
            cp_r.wait()
            cp_l.wait()
            prev = nxt
        for s in range(2):
            for d in range(2):
                prev[d][s].wait()
        cp_r = store_half(qr_ref, stage_r_ref, out_sem_r, (my - 2) % N_DEV, 0)
        cp_l = store_half(
            ql_ref, stage_l_ref, out_sem_l, (my + 2) % N_DEV, HALF
        )
        cp_r.wait()
        cp_l.wait()

        @functools.partial(
            pl.run_scoped, second_barrier=pltpu.SemaphoreType.REGULAR
        )
        def _(second_barrier):
            for nbr in (left, right):
                pl.semaphore_signal(
                    second_barrier, inc=1,
                    device_id=(nbr,), device_id_type=pl.DeviceIdType.MESH,
                )
            pl.semaphore_wait(second_barrier, 2)

    return pl.pallas_call(
        body,
        out_shape=jax.ShapeDtypeStruct((M, N_OUT), jnp.float32),
        in_specs=[
            pl.BlockSpec(memory_space=pl.ANY),
            pl.BlockSpec(memory_space=pltpu.VMEM),
        ],
        out_specs=pl.BlockSpec(memory_space=pl.ANY),
        scratch_shapes=[
            pltpu.VMEM((CHUNK, K_SHARD), jnp.float32),
            pltpu.VMEM((CHUNK, K_SHARD), jnp.bfloat16),
            pltpu.VMEM((K_SHARD, N_OUT), jnp.bfloat16),
            pltpu.VMEM((CHUNK, HALF), jnp.bfloat16),
            pltpu.VMEM((CHUNK, HALF), jnp.bfloat16),
            pltpu.VMEM((N_DEV - 1, CHUNK, HALF), jnp.bfloat16),
            pltpu.VMEM((N_DEV - 1, CHUNK, HALF), jnp.bfloat16),
            pltpu.VMEM((N_DEV, CHUNK, HALF), jnp.int8),
            pltpu.VMEM((N_DEV, CHUNK, HALF), jnp.int8),
            pltpu.VMEM((CHUNK, HALF), jnp.float32),
            pltpu.VMEM((CHUNK, HALF), jnp.float32),
            pltpu.VMEM((N_DEV, 8, 128), jnp.float32),
            pltpu.SemaphoreType.DMA,
            pltpu.SemaphoreType.DMA,
            pltpu.SemaphoreType.DMA,
            pltpu.SemaphoreType.DMA((N_DEV - 1, 2)),
            pltpu.SemaphoreType.DMA((N_DEV - 1, 2)),
            pltpu.SemaphoreType.DMA((N_DEV - 1, 2)),
            pltpu.SemaphoreType.DMA((N_DEV - 1, 2)),
            pltpu.SemaphoreType.DMA((N_DEV - 1,)),
            pltpu.SemaphoreType.DMA((N_DEV - 1,)),
            pltpu.SemaphoreType.DMA((N_DEV - 1, 2)),
            pltpu.SemaphoreType.DMA((N_DEV - 1, 2)),
            pltpu.SemaphoreType.DMA((N_DEV - 1, 2)),
            pltpu.SemaphoreType.DMA((N_DEV - 1, 2)),
        ],
        compiler_params=pltpu.CompilerParams(
            collective_id=0,
            vmem_limit_bytes=128 * 1024 * 1024,
        ),
    )(x, w_mat)


# device time: 145700 ns/iter; 1.0939x vs baseline; 1.0286x over previous
import functools

import jax
import jax.numpy as jnp
from jax import lax
from jax.experimental import pallas as pl
from jax.experimental.pallas import tpu as pltpu

N_DEV = 4
M = 4096
K_SHARD = 1024
N_OUT = 2048
HALF = N_OUT // 2
CHUNK = M // N_DEV
SUB = CHUNK // 2


def kernel(x, w_mat):
    def body(
        x_hbm,
        w_ref,
        out_hbm,
        xs_ref,
        xsb_ref,
        wb_ref,
        acc_r_ref,
        acc_l_ref,
        rsr_ref,
        rsl_ref,
        qr_ref,
        ql_ref,
        stage_r_ref,
        stage_l_ref,
        amax_ref,
        copy_sem,
        out_sem_r,
        out_sem_l,
        rs_send_r,
        rs_recv_r,
        rs_send_l,
        rs_recv_l,
        ax_send,
        ax_recv,
        ag_send_r,
        ag_recv_r,
        ag_send_l,
        ag_recv_l,
    ):
        my = lax.axis_index("i")
        right = (my + 1) % N_DEV
        left = (my + N_DEV - 1) % N_DEV

        barrier_sem = pltpu.get_barrier_semaphore()
        for nbr in (left, right):
            pl.semaphore_signal(
                barrier_sem, inc=1,
                device_id=(nbr,), device_id_type=pl.DeviceIdType.MESH,
            )

        wb_ref[...] = w_ref[...].astype(jnp.bfloat16)

        def load_x(c):
            cp = pltpu.make_async_copy(
                x_hbm.at[pl.ds(c * CHUNK, CHUNK), :], xs_ref, copy_sem
            )
            cp.start()
            cp.wait()
            xsb_ref[...] = xs_ref[...].astype(jnp.bfloat16)

        def dot_half(lo):
            return jnp.dot(
                xsb_ref[...],
                wb_ref[:, lo : lo + HALF],
                preferred_element_type=jnp.float32,
            )

        def rs_rdma(h, s, acc, rs_ref_, send_sems, recv_sems, dev):
            rows = pl.ds(s * SUB, SUB)
            return pltpu.make_async_remote_copy(
                src_ref=acc.at[rows],
                dst_ref=rs_ref_.at[h, rows],
                send_sem=send_sems.at[h, s],
                recv_sem=recv_sems.at[h, s],
                device_id=(dev,),
                device_id_type=pl.DeviceIdType.MESH,
            )

        def rs_rdma_r(h, s):
            return rs_rdma(h, s, acc_r_ref, rsr_ref, rs_send_r, rs_recv_r, right)

        def rs_rdma_l(h, s):
            return rs_rdma(h, s, acc_l_ref, rsl_ref, rs_send_l, rs_recv_l, left)

        def dot_block(row_lo, col_lo):
            return jnp.dot(
                xsb_ref[row_lo : row_lo + SUB, :],
                wb_ref[:, col_lo : col_lo + HALF],
                preferred_element_type=jnp.float32,
            )

        load_x(my)
        acc_r_ref[0:SUB] = dot_block(0, 0).astype(jnp.bfloat16)
        pl.semaphore_wait(barrier_sem, 2)
        cur = [[rs_rdma_r(0, 0), None], [None, None]]
        cur[0][0].start()
        acc_l_ref[0:SUB] = dot_block(0, HALF).astype(jnp.bfloat16)
        cur[1][0] = rs_rdma_l(0, 0)
        cur[1][0].start()
        acc_r_ref[SUB:CHUNK] = dot_block(SUB, 0).astype(jnp.bfloat16)
        cur[0][1] = rs_rdma_r(0, 1)
        cur[0][1].start()
        acc_l_ref[SUB:CHUNK] = dot_block(SUB, HALF).astype(jnp.bfloat16)
        cur[1][1] = rs_rdma_l(0, 1)
        cur[1][1].start()

        for h in range(N_DEV - 1):
            load_x((my - h - 1) % N_DEV)
            p_r = dot_half(0)
            if h != 1:
                load_x((my + h + 1) % N_DEV)
            p_l = dot_half(HALF)
            ps = [p_r, p_l]
            accs = [acc_r_ref, acc_l_ref]
            rs_refs = [rsr_ref, rsl_ref]
            mk = [rs_rdma_r, rs_rdma_l]
            nxt = [[None, None], [None, None]]
            last = h + 1 == N_DEV - 1
            maxes = []
            for s in range(2):
                rows = slice(s * SUB, (s + 1) * SUB)
                for d in range(2):
                    cur[d][s].wait()
                    v = rs_refs[d][h, rows].astype(jnp.float32) + ps[d][rows]
                    accs[d][rows] = v.astype(jnp.bfloat16)
                    if last:
                        maxes.append(jnp.max(jnp.abs(v)))
                    else:
                        nxt[d][s] = mk[d](h + 1, s)
                        nxt[d][s].start()
            cur = nxt

        own_r = (my + 1) % N_DEV
        own_l = (my + N_DEV - 1) % N_DEV

        my_max = functools.reduce(jnp.maximum, maxes)
        amax_ref[pl.ds(my, 1)] = jnp.full((1, 8, 128), my_max, jnp.float32)
        ax_rdmas = []
        for k in range(1, N_DEV):
            rdma = pltpu.make_async_remote_copy(
                src_ref=amax_ref.at[pl.ds(my, 1)],
                dst_ref=amax_ref.at[pl.ds(my, 1)],
                send_sem=ax_send.at[k - 1],
                recv_sem=ax_recv.at[k - 1],
                device_id=((my + k) % N_DEV,),
                device_id_type=pl.DeviceIdType.MESH,
            )
            rdma.start()
            ax_rdmas.append(rdma)
        for rdma in ax_rdmas:
            rdma.wait()
        amax = jnp.max(amax_ref[...])
        scale = amax / 127.0

        def quantize(v):
            return jnp.clip(jnp.round(v / scale), -127.0, 127.0).astype(
                jnp.int8
            )

        owns = [own_r, own_l]
        qs = [qr_ref, ql_ref]

        def store_half(q_ref_, stage_ref, sem, c, col_lo):
            stage_ref[...] = q_ref_[pl.ds(c, 1)][0].astype(jnp.float32) * scale
            cp = pltpu.make_async_copy(
                stage_ref,
                out_hbm.at[pl.ds(c * CHUNK, CHUNK), pl.ds(col_lo, HALF)],
                sem,
            )
            cp.start()
            return cp

        def ag_rdma(d, g, s):
            c = (my + 1 - g) % N_DEV if d == 0 else (my - 1 + g) % N_DEV
            q = qr_ref if d == 0 else ql_ref
            ss = ag_send_r if d == 0 else ag_send_l
            rs = ag_recv_r if d == 0 else ag_recv_l
            return pltpu.make_async_remote_copy(
                src_ref=q.at[pl.ds(c, 1), pl.ds(s * SUB, SUB)],
                dst_ref=q.at[pl.ds(c, 1), pl.ds(s * SUB, SUB)],
                send_sem=ss.at[g, s],
                recv_sem=rs.at[g, s],
                device_id=(right if d == 0 else left,),
                device_id_type=pl.DeviceIdType.MESH,
            )

        prev = [[None, None], [None, None]]
        for s in range(2):
            rows = pl.ds(s * SUB, SUB)
            for d in range(2):
                qs[d][pl.ds(owns[d], 1), rows] = quantize(
                    accs[d][s * SUB : (s + 1) * SUB].astype(jnp.float32)
                )[None]
                prev[d][s] = ag_rdma(d, 0, s)
                prev[d][s].start()
        cp_r = store_half(qr_ref, stage_r_ref, out_sem_r, own_r, 0)
        cp_l = store_half(ql_ref, stage_l_ref, out_sem_l, own_l, HALF)
        cp_r.wait()
        cp_l.wait()
        for g in range(1, N_DEV - 1):
            nxt = [[None, None], [None, None]]
            for s in range(2):
                for d in range(2):
                    prev[d][s].wait()
                    nxt[d][s] = ag_rdma(d, g, s)
                    nxt[d][s].start()
            cp_r = store_half(
                qr_ref, stage_r_ref, out_sem_r, (my - g + 1) % N_DEV, 0
            )
            cp_l = store_half(
                ql_ref, stage_l_ref, out_sem_l, (my + g - 1) % N_DEV, HALF
            )
            cp_r.wait()
            cp_l.wait()
            prev = nxt
        def store_sub(q_ref_, stage_ref, sem, c, col_lo, s):
            rows = pl.ds(s * SUB, SUB)
            stage_ref[rows] = (
                q_ref_[pl.ds(c, 1), rows][0].astype(jnp.float32) * scale
            )
            cp = pltpu.make_async_copy(
                stage_ref.at[rows],
                out_hbm.at[
                    pl.ds(c * CHUNK + s * SUB, SUB), pl.ds(col_lo, HALF)
                ],
                sem,
            )
            cp.start()
            return cp

        stages = [stage_r_ref, stage_l_ref]
        osems = [out_sem_r, out_sem_l]
        cols = [0, HALF]
        final_cs = [(my - 2) % N_DEV, (my + 2) % N_DEV]
        cps = []
        for s in range(2):
            for d in range(2):
                prev[d][s].wait()
                cps.append(
                    store_sub(qs[d], stages[d], osems[d], final_cs[d], cols[d], s)
                )
        for cp in cps:
            cp.wait()

        @functools.partial(
            pl.run_scoped, second_barrier=pltpu.SemaphoreType.REGULAR
        )
        def _(second_barrier):
            for nbr in (left, right):
                pl.semaphore_signal(
                    second_barrier, inc=1,
                    device_id=(nbr,), device_id_type=pl.DeviceIdType.MESH,
                )
            pl.semaphore_wait(second_barrier, 2)

    return pl.pallas_call(
        body,
        out_shape=jax.ShapeDtypeStruct((M, N_OUT), jnp.float32),
        in_specs=[
            pl.BlockSpec(memory_space=pl.ANY),
            pl.BlockSpec(memory_space=pltpu.VMEM),
        ],
        out_specs=pl.BlockSpec(memory_space=pl.ANY),
        scratch_shapes=[
            pltpu.VMEM((CHUNK, K_SHARD), jnp.float32),
            pltpu.VMEM((CHUNK, K_SHARD), jnp.bfloat16),
            pltpu.VMEM((K_SHARD, N_OUT), jnp.bfloat16),
            pltpu.VMEM((CHUNK, HALF), jnp.bfloat16),
            pltpu.VMEM((CHUNK, HALF), jnp.bfloat16),
            pltpu.VMEM((N_DEV - 1, CHUNK, HALF), jnp.bfloat16),
            pltpu.VMEM((N_DEV - 1, CHUNK, HALF), jnp.bfloat16),
            pltpu.VMEM((N_DEV, CHUNK, HALF), jnp.int8),
            pltpu.VMEM((N_DEV, CHUNK, HALF), jnp.int8),
            pltpu.VMEM((CHUNK, HALF), jnp.float32),
            pltpu.VMEM((CHUNK, HALF), jnp.float32),
            pltpu.VMEM((N_DEV, 8, 128), jnp.float32),
            pltpu.SemaphoreType.DMA,
            pltpu.SemaphoreType.DMA,
            pltpu.SemaphoreType.DMA,
            pltpu.SemaphoreType.DMA((N_DEV - 1, 2)),
            pltpu.SemaphoreType.DMA((N_DEV - 1, 2)),
            pltpu.SemaphoreType.DMA((N_DEV - 1, 2)),
            pltpu.SemaphoreType.DMA((N_DEV - 1, 2)),
            pltpu.SemaphoreType.DMA((N_DEV - 1,)),
            pltpu.SemaphoreType.DMA((N_DEV - 1,)),
            pltpu.SemaphoreType.DMA((N_DEV - 1, 2)),
            pltpu.SemaphoreType.DMA((N_DEV - 1, 2)),
            pltpu.SemaphoreType.DMA((N_DEV - 1, 2)),
            pltpu.SemaphoreType.DMA((N_DEV - 1, 2)),
        ],
        compiler_params=pltpu.CompilerParams(
            collective_id=0,
            vmem_limit_bytes=128 * 1024 * 1024,
        ),
    )(x, w_mat)


# device time: 143546 ns/iter; 1.1104x vs baseline; 1.0150x over previous
import functools

import jax
import jax.numpy as jnp
from jax import lax
from jax.experimental import pallas as pl
from jax.experimental.pallas import tpu as pltpu

N_DEV = 4
M = 4096
K_SHARD = 1024
N_OUT = 2048
HALF = N_OUT // 2
CHUNK = M // N_DEV
NS = 4
SUB = CHUNK // NS


def kernel(x, w_mat):
    def body(
        x_hbm,
        w_ref,
        out_hbm,
        xs_ref,
        xsb_ref,
        wb_ref,
        acc_r_ref,
        acc_l_ref,
        rsr_ref,
        rsl_ref,
        qr_ref,
        ql_ref,
        stage_r_ref,
        stage_l_ref,
        amax_ref,
        copy_sem,
        out_sem_r,
        out_sem_l,
        rs_send_r,
        rs_recv_r,
        rs_send_l,
        rs_recv_l,
        ax_send,
        ax_recv,
        ag_send_r,
        ag_recv_r,
        ag_send_l,
        ag_recv_l,
    ):
        my = lax.axis_index("i")
        right = (my + 1) % N_DEV
        left = (my + N_DEV - 1) % N_DEV

        barrier_sem = pltpu.get_barrier_semaphore()
        for nbr in (left, right):
            pl.semaphore_signal(
                barrier_sem, inc=1,
                device_id=(nbr,), device_id_type=pl.DeviceIdType.MESH,
            )

        wb_ref[...] = w_ref[...].astype(jnp.bfloat16)

        def load_x(c):
            cp = pltpu.make_async_copy(
                x_hbm.at[pl.ds(c * CHUNK, CHUNK), :], xs_ref, copy_sem
            )
            cp.start()
            cp.wait()
            xsb_ref[...] = xs_ref[...].astype(jnp.bfloat16)

        def dot_half(lo):
            return jnp.dot(
                xsb_ref[...],
                wb_ref[:, lo : lo + HALF],
                preferred_element_type=jnp.float32,
            )

        def rs_rdma(h, s, acc, rs_ref_, send_sems, recv_sems, dev):
            rows = pl.ds(s * SUB, SUB)
            return pltpu.make_async_remote_copy(
                src_ref=acc.at[rows],
                dst_ref=rs_ref_.at[h, rows],
                send_sem=send_sems.at[h, s],
                recv_sem=recv_sems.at[h, s],
                device_id=(dev,),
                device_id_type=pl.DeviceIdType.MESH,
            )

        def rs_rdma_r(h, s):
            return rs_rdma(h, s, acc_r_ref, rsr_ref, rs_send_r, rs_recv_r, right)

        def rs_rdma_l(h, s):
            return rs_rdma(h, s, acc_l_ref, rsl_ref, rs_send_l, rs_recv_l, left)

        def dot_block(row_lo, col_lo):
            return jnp.dot(
                xsb_ref[row_lo : row_lo + SUB, :],
                wb_ref[:, col_lo : col_lo + HALF],
                preferred_element_type=jnp.float32,
            )

        accs = [acc_r_ref, acc_l_ref]
        mk = [rs_rdma_r, rs_rdma_l]
        load_x(my)
        cur = [[None] * NS, [None] * NS]
        for s in range(NS):
            rows = slice(s * SUB, (s + 1) * SUB)
            for d in range(2):
                accs[d][rows] = dot_block(s * SUB, d * HALF).astype(
                    jnp.bfloat16
                )
                if s == 0 and d == 0:
                    pl.semaphore_wait(barrier_sem, 2)
                cur[d][s] = mk[d](0, s)
                cur[d][s].start()

        for h in range(N_DEV - 1):
            load_x((my - h - 1) % N_DEV)
            p_r = dot_half(0)
            if h != 1:
                load_x((my + h + 1) % N_DEV)
            p_l = dot_half(HALF)
            ps = [p_r, p_l]
            rs_refs = [rsr_ref, rsl_ref]
            nxt = [[None] * NS, [None] * NS]
            last = h + 1 == N_DEV - 1
            maxes = []
            for s in range(NS):
                rows = slice(s * SUB, (s + 1) * SUB)
                for d in range(2):
                    cur[d][s].wait()
                    v = rs_refs[d][h, rows].astype(jnp.float32) + ps[d][rows]
                    accs[d][rows] = v.astype(jnp.bfloat16)
                    if last:
                        maxes.append(jnp.max(jnp.abs(v)))
                    else:
                        nxt[d][s] = mk[d](h + 1, s)
                        nxt[d][s].start()
            cur = nxt

        own_r = (my + 1) % N_DEV
        own_l = (my + N_DEV - 1) % N_DEV

        my_max = functools.reduce(jnp.maximum, maxes)
        amax_ref[pl.ds(my, 1)] = jnp.full((1, 8, 128), my_max, jnp.float32)
        ax_rdmas = []
        for k in range(1, N_DEV):
            rdma = pltpu.make_async_remote_copy(
                src_ref=amax_ref.at[pl.ds(my, 1)],
                dst_ref=amax_ref.at[pl.ds(my, 1)],
                send_sem=ax_send.at[k - 1],
                recv_sem=ax_recv.at[k - 1],
                device_id=((my + k) % N_DEV,),
                device_id_type=pl.DeviceIdType.MESH,
            )
            rdma.start()
            ax_rdmas.append(rdma)
        for rdma in ax_rdmas:
            rdma.wait()
        amax = jnp.max(amax_ref[...])
        scale = amax / 127.0

        def quantize(v):
            return jnp.clip(jnp.round(v / scale), -127.0, 127.0).astype(
                jnp.int8
            )

        owns = [own_r, own_l]
        qs = [qr_ref, ql_ref]

        def store_half(q_ref_, stage_ref, sem, c, col_lo):
            stage_ref[...] = q_ref_[pl.ds(c, 1)][0].astype(jnp.float32) * scale
            cp = pltpu.make_async_copy(
                stage_ref,
                out_hbm.at[pl.ds(c * CHUNK, CHUNK), pl.ds(col_lo, HALF)],
                sem,
            )
            cp.start()
            return cp

        def ag_rdma(d, g, s):
            c = (my + 1 - g) % N_DEV if d == 0 else (my - 1 + g) % N_DEV
            q = qr_ref if d == 0 else ql_ref
            ss = ag_send_r if d == 0 else ag_send_l
            rs = ag_recv_r if d == 0 else ag_recv_l
            return pltpu.make_async_remote_copy(
                src_ref=q.at[pl.ds(c, 1), pl.ds(s * SUB, SUB)],
                dst_ref=q.at[pl.ds(c, 1), pl.ds(s * SUB, SUB)],
                send_sem=ss.at[g, s],
                recv_sem=rs.at[g, s],
                device_id=(right if d == 0 else left,),
                device_id_type=pl.DeviceIdType.MESH,
            )

        prev = [[None] * NS, [None] * NS]
        for s in range(NS):
            rows = pl.ds(s * SUB, SUB)
            for d in range(2):
                qs[d][pl.ds(owns[d], 1), rows] = quantize(
                    accs[d][s * SUB : (s + 1) * SUB].astype(jnp.float32)
                )[None]
                prev[d][s] = ag_rdma(d, 0, s)
                prev[d][s].start()
        cp_r = store_half(qr_ref, stage_r_ref, out_sem_r, own_r, 0)
        cp_l = store_half(ql_ref, stage_l_ref, out_sem_l, own_l, HALF)
        cp_r.wait()
        cp_l.wait()
        for g in range(1, N_DEV - 1):
            nxt = [[None] * NS, [None] * NS]
            for s in range(NS):
                for d in range(2):
                    prev[d][s].wait()
                    nxt[d][s] = ag_rdma(d, g, s)
                    nxt[d][s].start()
            cp_r = store_half(
                qr_ref, stage_r_ref, out_sem_r, (my - g + 1) % N_DEV, 0
            )
            cp_l = store_half(
                ql_ref, stage_l_ref, out_sem_l, (my + g - 1) % N_DEV, HALF
            )
            cp_r.wait()
            cp_l.wait()
            prev = nxt
        def store_sub(q_ref_, stage_ref, sem, c, col_lo, s):
            rows = pl.ds(s * SUB, SUB)
            stage_ref[rows] = (
                q_ref_[pl.ds(c, 1), rows][0].astype(jnp.float32) * scale
            )
            cp = pltpu.make_async_copy(
                stage_ref.at[rows],
                out_hbm.at[
                    pl.ds(c * CHUNK + s * SUB, SUB), pl.ds(col_lo, HALF)
                ],
                sem,
            )
            cp.start()
            return cp

        stages = [stage_r_ref, stage_l_ref]
        osems = [out_sem_r, out_sem_l]
        cols = [0, HALF]
        final_cs = [(my - 2) % N_DEV, (my + 2) % N_DEV]
        cps = []
        for s in range(NS):
            for d in range(2):
                prev[d][s].wait()
                cps.append(
                    store_sub(qs[d], stages[d], osems[d], final_cs[d], cols[d], s)
                )
        for cp in cps:
            cp.wait()

        @functools.partial(
            pl.run_scoped, second_barrier=pltpu.SemaphoreType.REGULAR
        )
        def _(second_barrier):
            for nbr in (left, right):
                pl.semaphore_signal(
                    second_barrier, inc=1,
                    device_id=(nbr,), device_id_type=pl.DeviceIdType.MESH,
                )
            pl.semaphore_wait(second_barrier, 2)

    return pl.pallas_call(
        body,
        out_shape=jax.ShapeDtypeStruct((M, N_OUT), jnp.float32),
        in_specs=[
            pl.BlockSpec(memory_space=pl.ANY),
            pl.BlockSpec(memory_space=pltpu.VMEM),
        ],
        out_specs=pl.BlockSpec(memory_space=pl.ANY),
        scratch_shapes=[
            pltpu.VMEM((CHUNK, K_SHARD), jnp.float32),
            pltpu.VMEM((CHUNK, K_SHARD), jnp.bfloat16),
            pltpu.VMEM((K_SHARD, N_OUT), jnp.bfloat16),
            pltpu.VMEM((CHUNK, HALF), jnp.bfloat16),
            pltpu.VMEM((CHUNK, HALF), jnp.bfloat16),
            pltpu.VMEM((N_DEV - 1, CHUNK, HALF), jnp.bfloat16),
            pltpu.VMEM((N_DEV - 1, CHUNK, HALF), jnp.bfloat16),
            pltpu.VMEM((N_DEV, CHUNK, HALF), jnp.int8),
            pltpu.VMEM((N_DEV, CHUNK, HALF), jnp.int8),
            pltpu.VMEM((CHUNK, HALF), jnp.float32),
            pltpu.VMEM((CHUNK, HALF), jnp.float32),
            pltpu.VMEM((N_DEV, 8, 128), jnp.float32),
            pltpu.SemaphoreType.DMA,
            pltpu.SemaphoreType.DMA,
            pltpu.SemaphoreType.DMA,
            pltpu.SemaphoreType.DMA((N_DEV - 1, NS)),
            pltpu.SemaphoreType.DMA((N_DEV - 1, NS)),
            pltpu.SemaphoreType.DMA((N_DEV - 1, NS)),
            pltpu.SemaphoreType.DMA((N_DEV - 1, NS)),
            pltpu.SemaphoreType.DMA((N_DEV - 1,)),
            pltpu.SemaphoreType.DMA((N_DEV - 1,)),
            pltpu.SemaphoreType.DMA((N_DEV - 1, NS)),
            pltpu.SemaphoreType.DMA((N_DEV - 1, NS)),
            pltpu.SemaphoreType.DMA((N_DEV - 1, NS)),
            pltpu.SemaphoreType.DMA((N_DEV - 1, NS)),
        ],
        compiler_params=pltpu.CompilerParams(
            collective_id=0,
            vmem_limit_bytes=128 * 1024 * 1024,
        ),
    )(x, w_mat)
